# baseline (speedup 1.0000x reference)
import jax
import jax.numpy as jnp
from jax import lax
from jax.experimental import pallas as pl
from jax.experimental.pallas import tpu as pltpu
from jax.experimental.pallas import tpu_sc as plsc

_NW = 32
_CH = 256


def kernel(x, table):
    b, s = x.shape
    n = b * s
    d = table.shape[1]
    idx_flat = x.reshape(n)
    tab128 = jnp.pad(table, ((0, 0), (0, 128 - d)))
    per_w = n // _NW
    chunks = per_w // _CH
    mesh = plsc.VectorSubcoreMesh(core_axis_name="c", subcore_axis_name="s")

    @pl.kernel(
        out_type=jax.ShapeDtypeStruct((n, d), table.dtype),
        mesh=mesh,
        scratch_types=[
            pltpu.VMEM((_CH,), jnp.int32),
            pltpu.VMEM((_CH,), jnp.int32),
            pltpu.VMEM((_CH, 128), table.dtype),
            pltpu.VMEM((_CH, 128), table.dtype),
            pltpu.SemaphoreType.DMA,
            pltpu.SemaphoreType.DMA,
        ],
        compiler_params=pltpu.CompilerParams(use_tc_tiling_on_sc=False),
    )
    def gather_kernel(table_hbm, idx_hbm, out_hbm, i0, i1, r0, r1, s0, s1):
        wid = lax.axis_index("s") * 2 + lax.axis_index("c")
        wbase = wid * per_w
        ibufs = (i0, i1)
        rbufs = (r0, r1)
        sems = (s0, s1)

        @pl.loop(0, chunks, step=2)
        def _(j):
            for k in range(2):
                tbase = wbase + (j + k) * _CH

                @pl.when(j > 0)
                def _():
                    pltpu.make_async_copy(
                        rbufs[k].at[:, pl.ds(0, d)],
                        out_hbm.at[pl.ds(tbase, _CH)],
                        sems[k],
                    ).wait()

                pltpu.sync_copy(idx_hbm.at[pl.ds(tbase, _CH)], ibufs[k])
                pltpu.sync_copy(table_hbm.at[ibufs[k]], rbufs[k])
                pltpu.async_copy(
                    rbufs[k].at[:, pl.ds(0, d)],
                    out_hbm.at[pl.ds(tbase, _CH)],
                    sems[k],
                )

        for k in range(2):
            pltpu.make_async_copy(
                rbufs[k].at[:, pl.ds(0, d)],
                out_hbm.at[pl.ds(wbase, _CH)],
                sems[k],
            ).wait()

    return gather_kernel(tab128, idx_flat).reshape(b, s, d)


# speedup vs baseline: 1.3068x; 1.3068x over previous
import jax
import jax.numpy as jnp
from jax.experimental import pallas as pl
from jax.experimental.pallas import tpu as pltpu
from jax.experimental.pallas import tpu_sc as plsc

_WINDOW = 1024


def _embedding_gather(idx_flat, table, n, d):
    mesh = plsc.VectorSubcoreMesh(core_axis_name="c", subcore_axis_name="s")

    @pl.kernel(
        out_type=jax.ShapeDtypeStruct((n, d), table.dtype),
        mesh=mesh,
        compiler_params=pltpu.CompilerParams(use_tc_tiling_on_sc=False),
    )
    def gather_kernel(table_hbm, idx_hbm, out_hbm):
        def body(idx_vmem, out_vmem):
            pltpu.sync_copy(table_hbm.at[idx_vmem.at[0]], out_vmem)

        pltpu.emit_pipeline(
            body,
            grid=(n // _WINDOW,),
            in_specs=[
                pl.BlockSpec((1, _WINDOW), index_map=lambda i: (0, i)),
            ],
            out_specs=[
                pl.BlockSpec((_WINDOW, d), index_map=lambda i: (i, 0)),
            ],
            core_axis_name=("c", "s"),
            dimension_semantics=(pltpu.PARALLEL,),
        )(idx_hbm, out_hbm)

    return gather_kernel(table, idx_flat)


def kernel(x, table):
    b, s = x.shape
    n = b * s
    d = table.shape[1]
    idx_flat = x.reshape(1, n)
    out = _embedding_gather(idx_flat, table, n, d)
    return out.reshape(b, s, d)


# speedup vs baseline: 1.3082x; 1.0011x over previous
import jax
import jax.numpy as jnp
from jax.experimental import pallas as pl
from jax.experimental.pallas import tpu as pltpu
from jax.experimental.pallas import tpu_sc as plsc

_WINDOW = 1600


def _embedding_gather(idx_flat, table, n, d):
    mesh = plsc.VectorSubcoreMesh(core_axis_name="c", subcore_axis_name="s")

    @pl.kernel(
        out_type=jax.ShapeDtypeStruct((n, d), table.dtype),
        mesh=mesh,
        compiler_params=pltpu.CompilerParams(use_tc_tiling_on_sc=False),
    )
    def gather_kernel(table_hbm, idx_hbm, out_hbm):
        def body(idx_vmem, out_vmem):
            pltpu.sync_copy(table_hbm.at[idx_vmem.at[0]], out_vmem)

        pltpu.emit_pipeline(
            body,
            grid=(n // _WINDOW,),
            in_specs=[
                pl.BlockSpec((1, _WINDOW), index_map=lambda i: (0, i)),
            ],
            out_specs=[
                pl.BlockSpec((_WINDOW, d), index_map=lambda i: (i, 0)),
            ],
            core_axis_name=("c", "s"),
            dimension_semantics=(pltpu.PARALLEL,),
        )(idx_hbm, out_hbm)

    return gather_kernel(table, idx_flat)


def kernel(x, table):
    b, s = x.shape
    n = b * s
    d = table.shape[1]
    idx_flat = x.reshape(1, n)
    out = _embedding_gather(idx_flat, table, n, d)
    return out.reshape(b, s, d)
